# baseline (speedup 1.0000x reference)
import jax
import jax.numpy as jnp
from jax import lax
from jax.experimental import pallas as pl
from jax.experimental.pallas import tpu as pltpu
from jax.experimental.pallas import tpu_sc as plsc


NC = 2
NS = 16
NW = NC * NS
BPW = 2
NBUF = 4


def kernel(x, table):
    batch, hist = x.shape
    num_embeddings, embed_dim = table.shape
    num_indices = batch * hist
    hist_pad = (hist + 7) // 8 * 8
    b_per_w = batch * hist_pad // NW
    batch_per_w = batch // NW

    idx = jnp.pad(x, ((0, 0), (0, hist_pad - hist))).reshape(batch * hist_pad)

    padded = jnp.pad(table, ((0, 0), (0, 128 - embed_dim)))

    mesh = plsc.VectorSubcoreMesh(core_axis_name="c", subcore_axis_name="s")

    @pl.kernel(
        out_type=jax.ShapeDtypeStruct((batch * hist_pad, 128), table.dtype),
        mesh=mesh,
        scratch_types=[
            pltpu.VMEM((b_per_w,), jnp.int32),
            pltpu.VMEM((NBUF, BPW * hist_pad, 128), table.dtype),
            pltpu.SemaphoreType.DMA((NBUF,)),
            pltpu.SemaphoreType.DMA((NBUF,)),
        ],
    )
    def gather_kernel(table_hbm, idx_hbm, out_hbm, idx_v, rows_v, gsem, wsem):
        wid = lax.axis_index("s") * NC + lax.axis_index("c")
        base = wid * b_per_w
        bbase = wid * batch_per_w

        pltpu.sync_copy(idx_hbm.at[pl.ds(base, b_per_w)], idx_v)

        @pl.loop(0, batch_per_w, step=BPW * NBUF)
        def _(g):
            gathers = []
            for u in range(NBUF):
                @pl.when(g > 0)
                def _():
                    pltpu.make_async_copy(
                        rows_v.at[u],
                        out_hbm.at[pl.ds(bbase * hist_pad, BPW * hist_pad)],
                        wsem.at[u],
                    ).wait()

                off = (g + u * BPW) * hist_pad
                gathers.append(
                    pltpu.async_copy(
                        table_hbm.at[idx_v.at[pl.ds(off, BPW * hist_pad)]],
                        rows_v.at[u],
                        gsem.at[u],
                    )
                )
            for u in range(NBUF):
                gathers[u].wait()
                b = bbase + g + u * BPW
                pltpu.async_copy(
                    rows_v.at[u],
                    out_hbm.at[pl.ds(b * hist_pad, BPW * hist_pad)],
                    wsem.at[u],
                )

        for u in range(NBUF):
            pltpu.make_async_copy(
                rows_v.at[u],
                out_hbm.at[pl.ds(bbase * hist_pad, BPW * hist_pad)],
                wsem.at[u],
            ).wait()

    out = gather_kernel(padded, idx)
    return out.reshape(batch, hist_pad, 128)[:, :hist, :embed_dim]


# speedup vs baseline: 5.1814x; 5.1814x over previous
import jax
import jax.numpy as jnp
from jax import lax
from jax.experimental import pallas as pl
from jax.experimental.pallas import tpu as pltpu
from jax.experimental.pallas import tpu_sc as plsc


NC = 2
NS = 16
NW = NC * NS
BPW = 1
NBUF = 8


def kernel(x, table):
    batch, hist = x.shape
    num_embeddings, embed_dim = table.shape
    num_indices = batch * hist
    hist_pad = (hist + 7) // 8 * 8
    b_per_w = batch * hist_pad // NW
    batch_per_w = batch // NW

    idx = jnp.pad(x, ((0, 0), (0, hist_pad - hist))).reshape(batch * hist_pad)

    padded = jnp.pad(
        table.reshape(2, num_embeddings // 2, embed_dim),
        ((0, 0), (0, 0), (0, 128 - embed_dim)),
    ).reshape(num_embeddings, 128)

    mesh = plsc.VectorSubcoreMesh(core_axis_name="c", subcore_axis_name="s")

    @pl.kernel(
        out_type=jax.ShapeDtypeStruct((batch, hist_pad, 128), table.dtype),
        mesh=mesh,
        scratch_types=[
            pltpu.VMEM((b_per_w,), jnp.int32),
            pltpu.VMEM((NBUF, BPW * hist_pad, 128), table.dtype),
            pltpu.SemaphoreType.DMA((NBUF,)),
            pltpu.SemaphoreType.DMA((NBUF,)),
        ],
    )
    def gather_kernel(table_hbm, idx_hbm, out_hbm, idx_v, rows_v, gsem, wsem):
        wid = lax.axis_index("s") * NC + lax.axis_index("c")
        base = wid * b_per_w
        bbase = wid * batch_per_w

        pltpu.sync_copy(idx_hbm.at[pl.ds(base, b_per_w)], idx_v)

        @pl.loop(0, batch_per_w, step=BPW * NBUF)
        def _(g):
            gathers = []
            for u in range(NBUF):
                @pl.when(g > 0)
                def _():
                    for v in range(BPW):
                        pltpu.make_async_copy(
                            rows_v.at[u, pl.ds(v * hist_pad, hist_pad)],
                            out_hbm.at[bbase],
                            wsem.at[u],
                        ).wait()

                for v in range(BPW):
                    off = (g + u * BPW + v) * hist_pad
                    gathers.append(
                        pltpu.async_copy(
                            table_hbm.at[idx_v.at[pl.ds(off, hist)]],
                            rows_v.at[u, pl.ds(v * hist_pad, hist)],
                            gsem.at[u],
                        )
                    )
            for u in range(NBUF):
                for v in range(BPW):
                    gathers[u * BPW + v].wait()
                    b = bbase + g + u * BPW + v
                    pltpu.async_copy(
                        rows_v.at[u, pl.ds(v * hist_pad, hist_pad)],
                        out_hbm.at[b],
                        wsem.at[u],
                    )

        for u in range(NBUF):
            for v in range(BPW):
                pltpu.make_async_copy(
                    rows_v.at[u, pl.ds(v * hist_pad, hist_pad)],
                    out_hbm.at[bbase],
                    wsem.at[u],
                ).wait()

    out = gather_kernel(padded, idx)
    return out[:, :hist, :embed_dim]
